# baseline (speedup 1.0000x reference)
import functools

import jax
import jax.numpy as jnp
from jax import lax
from jax.experimental import pallas as pl
from jax.experimental.pallas import tpu as pltpu
from jax.experimental.pallas import tpu_sc as plsc

D = 64
B = 16384
NC = 2
NS = 16
NW = NC * NS
BPW = B // NW
CHUNK = 128
NCH = BPW // CHUNK
L = 16


def _mf_body(uE, iE, uB, iB, usr, itm, out,
             idx_u, idx_i, u_rows, i_rows, ub, ib, out_v, sem):
    wid = lax.axis_index("s") * NC + lax.axis_index("c")
    base = wid * BPW

    pltpu.sync_copy(usr.at[pl.ds(base, BPW)], idx_u)
    pltpu.sync_copy(itm.at[pl.ds(base, BPW)], idx_i)

    copies = []
    for k in range(NCH):
        sl = pl.ds(k * CHUNK, CHUNK)
        copies.append(pltpu.async_copy(uE.at[idx_u.at[sl]], u_rows.at[sl], sem))
        copies.append(pltpu.async_copy(iE.at[idx_i.at[sl]], i_rows.at[sl], sem))
        copies.append(pltpu.async_copy(uB.at[idx_u.at[sl]], ub.at[sl], sem))
        copies.append(pltpu.async_copy(iB.at[idx_i.at[sl]], ib.at[sl], sem))
    for c in copies:
        c.wait()

    lanes = lax.iota(jnp.int32, L)

    def group(g, carry):
        dots = jnp.zeros((L,), jnp.float32)
        for pp in range(L):
            p = g * L + pp
            acc = u_rows[p, pl.ds(0, L)] * i_rows[p, pl.ds(0, L)]
            for c in range(1, D // L):
                acc = acc + u_rows[p, pl.ds(c * L, L)] * i_rows[p, pl.ds(c * L, L)]
            dots = jnp.where(lanes == pp, jnp.sum(acc), dots)
        sl = pl.ds(g * L, L)
        out_v[sl] = dots + ub[sl] + ib[sl]
        return carry

    lax.fori_loop(0, BPW // L, group, 0)

    pltpu.sync_copy(out_v, out.at[pl.ds(base, BPW)])


def kernel(uEmbeds, iEmbeds, uBias, iBias, usr, itm):
    mesh = plsc.VectorSubcoreMesh(core_axis_name="c", subcore_axis_name="s")
    run = functools.partial(
        pl.kernel,
        mesh=mesh,
        out_type=jax.ShapeDtypeStruct((B,), jnp.float32),
        compiler_params=pltpu.CompilerParams(
            needs_layout_passes=False, use_tc_tiling_on_sc=False),
        scratch_types=[
            pltpu.VMEM((BPW,), jnp.int32),
            pltpu.VMEM((BPW,), jnp.int32),
            pltpu.VMEM((BPW, D), jnp.float32),
            pltpu.VMEM((BPW, D), jnp.float32),
            pltpu.VMEM((BPW,), jnp.float32),
            pltpu.VMEM((BPW,), jnp.float32),
            pltpu.VMEM((BPW,), jnp.float32),
            pltpu.SemaphoreType.DMA,
        ],
    )(_mf_body)
    return run(uEmbeds, iEmbeds, uBias, iBias, usr, itm)
